# baseline (speedup 1.0000x reference)
import functools

import jax
import jax.numpy as jnp
from jax import lax
from jax.experimental import pallas as pl
from jax.experimental.pallas import tpu as pltpu
from jax.experimental.pallas import tpu_sc as plsc

NC = 2
NS = 16
NW = NC * NS

N_NODES = 10000
N_ACC = 10240
ROWS_PER_TILE = N_ACC // NS


def _mesh():
    return plsc.VectorSubcoreMesh(
        core_axis_name="c", subcore_axis_name="s", num_cores=NC, num_subcores=NS
    )


@functools.partial(jax.jit, static_argnums=(1,))
def _sc_degree(dst2d, n_idx_rows_per_worker):
    KD = 8
    n_outer = n_idx_rows_per_worker // KD
    ones = jnp.ones((128, 128), jnp.float32)
    zeros = jnp.zeros((ROWS_PER_TILE, 128), jnp.float32)

    @functools.partial(
        pl.kernel,
        out_type=jax.ShapeDtypeStruct((NC, N_ACC, 128), jnp.float32),
        mesh=_mesh(),
        scratch_types=[
            pltpu.VMEM((KD, 128), jnp.int32),
            pltpu.VMEM((128, 128), jnp.float32),
            pltpu.VMEM_SHARED((N_ACC, 128), jnp.float32),
        ],
    )
    def deg_kernel(dst_hbm, ones_hbm, zeros_hbm, out_hbm, dst_v, ones_v, deg_sh):
        c = lax.axis_index("c")
        s = lax.axis_index("s")
        wid = c * NS + s
        pltpu.sync_copy(ones_hbm, ones_v)
        pltpu.sync_copy(zeros_hbm, deg_sh.at[pl.ds(s * ROWS_PER_TILE, ROWS_PER_TILE)])
        plsc.subcore_barrier()

        def body(i, carry):
            base = wid * n_idx_rows_per_worker + i * KD
            pltpu.sync_copy(dst_hbm.at[pl.ds(base, KD)], dst_v)
            for j in range(KD):
                pltpu.sync_copy(ones_v, deg_sh.at[dst_v.at[j]], add=True)
            return carry

        lax.fori_loop(0, n_outer, body, 0)
        plsc.subcore_barrier()
        pltpu.sync_copy(
            deg_sh.at[pl.ds(s * ROWS_PER_TILE, ROWS_PER_TILE)],
            out_hbm.at[c, pl.ds(s * ROWS_PER_TILE, ROWS_PER_TILE)],
        )

    return deg_kernel(dst2d, ones, zeros)


@functools.partial(jax.jit, static_argnums=(3, 4))
def _sc_aggregate(h, src2d, dst2d, d_feat, n_idx_rows_per_worker):
    G = 8
    NB = 2
    n_outer = n_idx_rows_per_worker // G
    zeros = jnp.zeros((ROWS_PER_TILE, d_feat), jnp.float32)

    @functools.partial(
        pl.kernel,
        out_type=jax.ShapeDtypeStruct((NC, N_ACC, d_feat), jnp.float32),
        mesh=_mesh(),
        scratch_types=[
            pltpu.VMEM((G, 128), jnp.int32),
            pltpu.VMEM((G, 128), jnp.int32),
            pltpu.VMEM((NB, 128, d_feat), jnp.float32),
            pltpu.VMEM_SHARED((N_ACC, d_feat), jnp.float32),
            [pltpu.SemaphoreType.DMA] * NB,
            [pltpu.SemaphoreType.DMA] * NB,
        ],
    )
    def agg_kernel(h_hbm, src_hbm, dst_hbm, zeros_hbm, out_hbm,
                   src_v, dst_v, rows_v, agg_sh, gsems, ssems):
        c = lax.axis_index("c")
        s = lax.axis_index("s")
        wid = c * NS + s
        pltpu.sync_copy(zeros_hbm, agg_sh.at[pl.ds(s * ROWS_PER_TILE, ROWS_PER_TILE)])
        plsc.subcore_barrier()

        def body(i, carry):
            base = wid * n_idx_rows_per_worker + i * G
            pltpu.sync_copy(src_hbm.at[pl.ds(base, G)], src_v)
            pltpu.sync_copy(dst_hbm.at[pl.ds(base, G)], dst_v)
            gcps = [
                pltpu.async_copy(h_hbm.at[src_v.at[j]], rows_v.at[j], gsems[j])
                for j in range(NB)
            ]
            scps = [None] * NB
            for j in range(G):
                b = j % NB
                gcps[b].wait()
                scps[b] = pltpu.async_copy(rows_v.at[b], agg_sh.at[dst_v.at[j]],
                                           ssems[b], add=True)
                pj = j - 1
                if pj >= 0 and pj + NB < G:
                    pb = pj % NB
                    scps[pb].wait()
                    gcps[pb] = pltpu.async_copy(
                        h_hbm.at[src_v.at[pj + NB]], rows_v.at[pb], gsems[pb])
            for b in range(NB):
                if scps[b] is not None:
                    scps[b].wait()
            return carry

        lax.fori_loop(0, n_outer, body, 0)
        plsc.subcore_barrier()
        pltpu.sync_copy(
            agg_sh.at[pl.ds(s * ROWS_PER_TILE, ROWS_PER_TILE)],
            out_hbm.at[c, pl.ds(s * ROWS_PER_TILE, ROWS_PER_TILE)],
        )

    return agg_kernel(h, src2d, dst2d, zeros)


def _tc_first(deg_parts, x, w):
    n, d = x.shape

    def body(deg_ref, x_ref, w_ref, norm_ref, o_ref):
        dg = deg_ref[0, :n, 0:1] + deg_ref[1, :n, 0:1]
        nrm = jnp.where(dg > 0, 1.0 / jnp.sqrt(jnp.maximum(dg, 1.0)), 0.0)
        nrm2d = jnp.broadcast_to(nrm, (n, d))
        norm_ref[...] = nrm2d
        o_ref[...] = jnp.dot(x_ref[...], w_ref[...],
                             preferred_element_type=jnp.float32) * nrm2d

    return pl.pallas_call(
        body,
        out_shape=(
            jax.ShapeDtypeStruct((n, d), jnp.float32),
            jax.ShapeDtypeStruct((n, w.shape[1]), jnp.float32),
        ),
    )(deg_parts, x, w)


def _tc_post_and_next(parts, norm2d, b, w_next):
    n, d = norm2d.shape
    d_out = w_next.shape[1]

    def body(p_ref, norm_ref, b_ref, w_ref, o_ref):
        nrm = norm_ref[...]
        z = (p_ref[0, :n, :] + p_ref[1, :n, :]) * nrm + b_ref[...][None, :]
        z = jnp.maximum(z, 0.0)
        mu = jnp.mean(z)
        zc = z - mu
        var = jnp.mean(zc * zc)
        zn = zc / jnp.sqrt(var + 1e-5)
        o_ref[...] = jnp.dot(zn * nrm, w_ref[...],
                             preferred_element_type=jnp.float32)

    return pl.pallas_call(
        body,
        out_shape=jax.ShapeDtypeStruct((n, d_out), jnp.float32),
    )(parts, norm2d, b, w_next)


def _tc_final(parts, norm2d, b):
    n = norm2d.shape[0]
    d = b.shape[0]

    def body(p_ref, norm_ref, b_ref, o_ref):
        nrm = norm_ref[...][:, :d]
        o_ref[...] = (p_ref[0, :n, :d] + p_ref[1, :n, :d]) * nrm + b_ref[...][None, :]

    return pl.pallas_call(
        body,
        out_shape=jax.ShapeDtypeStruct((n, d), jnp.float32),
    )(parts, norm2d, b)


def kernel(features, edge_index, W0, b0, W1, b1, W2, b2, num_bits, num_grad_bits):
    n, _ = features.shape
    e = edge_index.shape[1]

    epw = ((e + NW - 1) // NW + 1023) // 1024 * 1024
    e_pad = epw * NW
    rows_per_worker = epw // 128
    pad_idx = jnp.arange(e_pad - e, dtype=jnp.int32)
    src = jnp.concatenate(
        [edge_index[0], pad_idx % n]
    ).reshape(e_pad // 128, 128)
    dst = jnp.concatenate(
        [edge_index[1], n + pad_idx % (N_ACC - n)]
    ).reshape(e_pad // 128, 128)

    deg_parts = _sc_degree(dst, rows_per_worker)

    norm2d, m0 = _tc_first(deg_parts, features, W0)
    p0 = _sc_aggregate(m0, src, dst, 128, rows_per_worker)

    m1 = _tc_post_and_next(p0, norm2d, b0, W1)
    p1 = _sc_aggregate(m1, src, dst, 128, rows_per_worker)

    w2p = jnp.concatenate([W2, jnp.zeros((W2.shape[0], 128 - W2.shape[1]),
                                         jnp.float32)], axis=1)
    m2 = _tc_post_and_next(p1, norm2d, b1, w2p)
    p2 = _sc_aggregate(m2, src, dst, 128, rows_per_worker)

    return _tc_final(p2, norm2d, b2)


# speedup vs baseline: 1.1023x; 1.1023x over previous
import functools

import jax
import jax.numpy as jnp
from jax import lax
from jax.experimental import pallas as pl
from jax.experimental.pallas import tpu as pltpu
from jax.experimental.pallas import tpu_sc as plsc

NC = 2
NS = 16
NW = NC * NS

N_NODES = 10000
N_ACC = 10240
ROWS_PER_TILE = N_ACC // NS


def _mesh():
    return plsc.VectorSubcoreMesh(
        core_axis_name="c", subcore_axis_name="s", num_cores=NC, num_subcores=NS
    )


@functools.partial(jax.jit, static_argnums=(1,))
def _sc_degree(dst2d, n_idx_rows_per_worker):
    KD = 8
    n_outer = n_idx_rows_per_worker // KD
    ones = jnp.ones((128, 128), jnp.float32)
    zeros = jnp.zeros((ROWS_PER_TILE, 128), jnp.float32)

    @functools.partial(
        pl.kernel,
        out_type=jax.ShapeDtypeStruct((NC, N_ACC, 128), jnp.float32),
        mesh=_mesh(),
        scratch_types=[
            pltpu.VMEM((KD, 128), jnp.int32),
            pltpu.VMEM((128, 128), jnp.float32),
            pltpu.VMEM_SHARED((N_ACC, 128), jnp.float32),
        ],
    )
    def deg_kernel(dst_hbm, ones_hbm, zeros_hbm, out_hbm, dst_v, ones_v, deg_sh):
        c = lax.axis_index("c")
        s = lax.axis_index("s")
        wid = c * NS + s
        pltpu.sync_copy(ones_hbm, ones_v)
        pltpu.sync_copy(zeros_hbm, deg_sh.at[pl.ds(s * ROWS_PER_TILE, ROWS_PER_TILE)])
        plsc.subcore_barrier()

        def body(i, carry):
            base = wid * n_idx_rows_per_worker + i * KD
            pltpu.sync_copy(dst_hbm.at[pl.ds(base, KD)], dst_v)
            for j in range(KD):
                pltpu.sync_copy(ones_v, deg_sh.at[dst_v.at[j]], add=True)
            return carry

        lax.fori_loop(0, n_outer, body, 0)
        plsc.subcore_barrier()
        pltpu.sync_copy(
            deg_sh.at[pl.ds(s * ROWS_PER_TILE, ROWS_PER_TILE)],
            out_hbm.at[c, pl.ds(s * ROWS_PER_TILE, ROWS_PER_TILE)],
        )

    return deg_kernel(dst2d, ones, zeros)


@functools.partial(jax.jit, static_argnums=(3, 4))
def _sc_aggregate(h, src2d, dst2d, d_feat, n_idx_rows_per_worker):
    G = 8
    NB = 2
    n_outer = n_idx_rows_per_worker // G
    zeros = jnp.zeros((ROWS_PER_TILE, d_feat), jnp.float32)

    @functools.partial(
        pl.kernel,
        out_type=jax.ShapeDtypeStruct((NC, N_ACC, d_feat), jnp.float32),
        mesh=_mesh(),
        scratch_types=[
            pltpu.VMEM((G, 128), jnp.int32),
            pltpu.VMEM((G, 128), jnp.int32),
            pltpu.VMEM((NB, 128, d_feat), jnp.float32),
            pltpu.VMEM_SHARED((N_ACC, d_feat), jnp.float32),
            [pltpu.SemaphoreType.DMA] * NB,
            [pltpu.SemaphoreType.DMA] * NB,
        ],
    )
    def agg_kernel(h_hbm, src_hbm, dst_hbm, zeros_hbm, out_hbm,
                   src_v, dst_v, rows_v, agg_sh, gsems, ssems):
        c = lax.axis_index("c")
        s = lax.axis_index("s")
        wid = c * NS + s
        pltpu.sync_copy(zeros_hbm, agg_sh.at[pl.ds(s * ROWS_PER_TILE, ROWS_PER_TILE)])
        plsc.subcore_barrier()

        def body(i, carry):
            base = wid * n_idx_rows_per_worker + i * G
            pltpu.sync_copy(src_hbm.at[pl.ds(base, G)], src_v)
            pltpu.sync_copy(dst_hbm.at[pl.ds(base, G)], dst_v)
            gcps = [
                pltpu.async_copy(h_hbm.at[src_v.at[j]], rows_v.at[j], gsems[j])
                for j in range(NB)
            ]
            for j in range(G):
                b = j % NB
                gcps[b].wait()
                pltpu.sync_copy(rows_v.at[b], agg_sh.at[dst_v.at[j]],
                                add=True)
                if j + NB < G:
                    gcps[b] = pltpu.async_copy(
                        h_hbm.at[src_v.at[j + NB]], rows_v.at[b], gsems[b])
            return carry

        lax.fori_loop(0, n_outer, body, 0)
        plsc.subcore_barrier()
        pltpu.sync_copy(
            agg_sh.at[pl.ds(s * ROWS_PER_TILE, ROWS_PER_TILE)],
            out_hbm.at[c, pl.ds(s * ROWS_PER_TILE, ROWS_PER_TILE)],
        )

    return agg_kernel(h, src2d, dst2d, zeros)


def _tc_first(deg_parts, x, w):
    n, d = x.shape

    def body(deg_ref, x_ref, w_ref, norm_ref, o_ref):
        dg = deg_ref[0, :n, 0:1] + deg_ref[1, :n, 0:1]
        nrm = jnp.where(dg > 0, 1.0 / jnp.sqrt(jnp.maximum(dg, 1.0)), 0.0)
        nrm2d = jnp.broadcast_to(nrm, (n, d))
        norm_ref[...] = nrm2d
        o_ref[...] = jnp.dot(x_ref[...], w_ref[...],
                             preferred_element_type=jnp.float32) * nrm2d

    return pl.pallas_call(
        body,
        out_shape=(
            jax.ShapeDtypeStruct((n, d), jnp.float32),
            jax.ShapeDtypeStruct((n, w.shape[1]), jnp.float32),
        ),
    )(deg_parts, x, w)


def _tc_post_and_next(parts, norm2d, b, w_next):
    n, d = norm2d.shape
    d_out = w_next.shape[1]

    def body(p_ref, norm_ref, b_ref, w_ref, o_ref):
        nrm = norm_ref[...]
        z = (p_ref[0, :n, :] + p_ref[1, :n, :]) * nrm + b_ref[...][None, :]
        z = jnp.maximum(z, 0.0)
        mu = jnp.mean(z)
        zc = z - mu
        var = jnp.mean(zc * zc)
        zn = zc / jnp.sqrt(var + 1e-5)
        o_ref[...] = jnp.dot(zn * nrm, w_ref[...],
                             preferred_element_type=jnp.float32)

    return pl.pallas_call(
        body,
        out_shape=jax.ShapeDtypeStruct((n, d_out), jnp.float32),
    )(parts, norm2d, b, w_next)


def _tc_final(parts, norm2d, b):
    n = norm2d.shape[0]
    d = b.shape[0]

    def body(p_ref, norm_ref, b_ref, o_ref):
        nrm = norm_ref[...][:, :d]
        o_ref[...] = (p_ref[0, :n, :d] + p_ref[1, :n, :d]) * nrm + b_ref[...][None, :]

    return pl.pallas_call(
        body,
        out_shape=jax.ShapeDtypeStruct((n, d), jnp.float32),
    )(parts, norm2d, b)


def kernel(features, edge_index, W0, b0, W1, b1, W2, b2, num_bits, num_grad_bits):
    n, _ = features.shape
    e = edge_index.shape[1]

    epw = ((e + NW - 1) // NW + 1023) // 1024 * 1024
    e_pad = epw * NW
    rows_per_worker = epw // 128
    pad_idx = jnp.arange(e_pad - e, dtype=jnp.int32)
    src = jnp.concatenate(
        [edge_index[0], pad_idx % n]
    ).reshape(e_pad // 128, 128)
    dst = jnp.concatenate(
        [edge_index[1], n + pad_idx % (N_ACC - n)]
    ).reshape(e_pad // 128, 128)

    deg_parts = _sc_degree(dst, rows_per_worker)

    norm2d, m0 = _tc_first(deg_parts, features, W0)
    p0 = _sc_aggregate(m0, src, dst, 128, rows_per_worker)

    m1 = _tc_post_and_next(p0, norm2d, b0, W1)
    p1 = _sc_aggregate(m1, src, dst, 128, rows_per_worker)

    w2p = jnp.concatenate([W2, jnp.zeros((W2.shape[0], 128 - W2.shape[1]),
                                         jnp.float32)], axis=1)
    m2 = _tc_post_and_next(p1, norm2d, b1, w2p)
    p2 = _sc_aggregate(m2, src, dst, 128, rows_per_worker)

    return _tc_final(p2, norm2d, b2)


# speedup vs baseline: 1.2493x; 1.1334x over previous
import functools

import jax
import jax.numpy as jnp
from jax import lax
from jax.experimental import pallas as pl
from jax.experimental.pallas import tpu as pltpu
from jax.experimental.pallas import tpu_sc as plsc

NC = 2
NS = 16
NW = NC * NS

N_NODES = 10000
N_ACC = 10240
ROWS_PER_TILE = N_ACC // NS


def _mesh():
    return plsc.VectorSubcoreMesh(
        core_axis_name="c", subcore_axis_name="s", num_cores=NC, num_subcores=NS
    )


DEG_ROWS = N_ACC // 128


@functools.partial(jax.jit, static_argnums=(1,))
def _sc_degree(dst2d, n_idx_rows_per_worker):
    rpw = n_idx_rows_per_worker
    npt = N_ACC // NS

    @functools.partial(
        pl.kernel,
        out_type=jax.ShapeDtypeStruct((NC, N_ACC), jnp.float32),
        mesh=_mesh(),
        compiler_params=pltpu.CompilerParams(needs_layout_passes=False),
        scratch_types=[
            pltpu.VMEM((rpw, 128), jnp.int32),
            pltpu.VMEM((N_ACC,), jnp.float32),
            pltpu.VMEM((NS, npt), jnp.float32),
            pltpu.VMEM_SHARED((NS, N_ACC), jnp.float32),
        ],
    )
    def deg_kernel(dst_hbm, out_hbm, dst_v, deg_loc, red_v, stage_sh):
        c = lax.axis_index("c")
        s = lax.axis_index("s")
        wid = c * NS + s
        zeros16 = jnp.zeros((16,), jnp.float32)
        ones16 = jnp.ones((16,), jnp.float32)

        def zero_body(i, carry):
            deg_loc[pl.ds(pl.multiple_of(i * 16, 16), 16)] = zeros16
            return carry

        lax.fori_loop(0, N_ACC // 16, zero_body, 0)
        pltpu.sync_copy(dst_hbm.at[pl.ds(wid * rpw, rpw)], dst_v)

        def body(i, carry):
            d16 = dst_v[i // 8, pl.ds(pl.multiple_of((i % 8) * 16, 16), 16)]
            plsc.addupdate_scatter(deg_loc, [d16], ones16)
            return carry

        lax.fori_loop(0, rpw * 8, body, 0)

        pltpu.sync_copy(deg_loc, stage_sh.at[s])
        plsc.subcore_barrier()
        pltpu.sync_copy(stage_sh.at[:, pl.ds(s * npt, npt)], red_v)

        def red_body(i, carry):
            off = pl.ds(pl.multiple_of(i * 16, 16), 16)
            acc = red_v[0, off]
            for k in range(1, NS):
                acc = acc + red_v[k, off]
            deg_loc[off] = acc
            return carry

        lax.fori_loop(0, npt // 16, red_body, 0)
        pltpu.sync_copy(deg_loc.at[pl.ds(0, npt)],
                        out_hbm.at[c, pl.ds(s * npt, npt)])

    return deg_kernel(dst2d)


@functools.partial(jax.jit, static_argnums=(3, 4))
def _sc_aggregate(h, src2d, dst2d, d_feat, n_idx_rows_per_worker):
    G = 8
    NB = 2
    n_outer = n_idx_rows_per_worker // G
    zeros = jnp.zeros((ROWS_PER_TILE, d_feat), jnp.float32)

    @functools.partial(
        pl.kernel,
        out_type=jax.ShapeDtypeStruct((NC, N_ACC, d_feat), jnp.float32),
        mesh=_mesh(),
        scratch_types=[
            pltpu.VMEM((G, 128), jnp.int32),
            pltpu.VMEM((G, 128), jnp.int32),
            pltpu.VMEM((NB, 128, d_feat), jnp.float32),
            pltpu.VMEM_SHARED((N_ACC, d_feat), jnp.float32),
            [pltpu.SemaphoreType.DMA] * NB,
            [pltpu.SemaphoreType.DMA] * NB,
        ],
    )
    def agg_kernel(h_hbm, src_hbm, dst_hbm, zeros_hbm, out_hbm,
                   src_v, dst_v, rows_v, agg_sh, gsems, ssems):
        c = lax.axis_index("c")
        s = lax.axis_index("s")
        wid = c * NS + s
        pltpu.sync_copy(zeros_hbm, agg_sh.at[pl.ds(s * ROWS_PER_TILE, ROWS_PER_TILE)])
        plsc.subcore_barrier()

        def body(i, carry):
            base = wid * n_idx_rows_per_worker + i * G
            pltpu.sync_copy(src_hbm.at[pl.ds(base, G)], src_v)
            pltpu.sync_copy(dst_hbm.at[pl.ds(base, G)], dst_v)
            gcps = [
                pltpu.async_copy(h_hbm.at[src_v.at[j]], rows_v.at[j], gsems[j])
                for j in range(NB)
            ]
            for j in range(G):
                b = j % NB
                gcps[b].wait()
                pltpu.sync_copy(rows_v.at[b], agg_sh.at[dst_v.at[j]],
                                add=True)
                if j + NB < G:
                    gcps[b] = pltpu.async_copy(
                        h_hbm.at[src_v.at[j + NB]], rows_v.at[b], gsems[b])
            return carry

        lax.fori_loop(0, n_outer, body, 0)
        plsc.subcore_barrier()
        pltpu.sync_copy(
            agg_sh.at[pl.ds(s * ROWS_PER_TILE, ROWS_PER_TILE)],
            out_hbm.at[c, pl.ds(s * ROWS_PER_TILE, ROWS_PER_TILE)],
        )

    return agg_kernel(h, src2d, dst2d, zeros)


def _tc_first(deg_nm, x, w):
    n, d = x.shape

    def body(deg_ref, x_ref, w_ref, norm_ref, o_ref):
        dg = deg_ref[:, 0:1] + deg_ref[:, 1:2]
        nrm = jnp.where(dg > 0, 1.0 / jnp.sqrt(jnp.maximum(dg, 1.0)), 0.0)
        nrm2d = jnp.broadcast_to(nrm, (n, d))
        norm_ref[...] = nrm2d
        o_ref[...] = jnp.dot(x_ref[...], w_ref[...],
                             preferred_element_type=jnp.float32) * nrm2d

    return pl.pallas_call(
        body,
        out_shape=(
            jax.ShapeDtypeStruct((n, d), jnp.float32),
            jax.ShapeDtypeStruct((n, w.shape[1]), jnp.float32),
        ),
    )(deg_nm, x, w)


def _tc_post_and_next(parts, norm2d, b, w_next):
    n, d = norm2d.shape
    d_out = w_next.shape[1]

    def body(p_ref, norm_ref, b_ref, w_ref, o_ref):
        nrm = norm_ref[...]
        z = (p_ref[0, :n, :] + p_ref[1, :n, :]) * nrm + b_ref[...][None, :]
        z = jnp.maximum(z, 0.0)
        mu = jnp.mean(z)
        zc = z - mu
        var = jnp.mean(zc * zc)
        zn = zc / jnp.sqrt(var + 1e-5)
        o_ref[...] = jnp.dot(zn * nrm, w_ref[...],
                             preferred_element_type=jnp.float32)

    return pl.pallas_call(
        body,
        out_shape=jax.ShapeDtypeStruct((n, d_out), jnp.float32),
    )(parts, norm2d, b, w_next)


def _tc_final(parts, norm2d, b):
    n = norm2d.shape[0]
    d = b.shape[0]

    def body(p_ref, norm_ref, b_ref, o_ref):
        nrm = norm_ref[...][:, :d]
        o_ref[...] = (p_ref[0, :n, :d] + p_ref[1, :n, :d]) * nrm + b_ref[...][None, :]

    return pl.pallas_call(
        body,
        out_shape=jax.ShapeDtypeStruct((n, d), jnp.float32),
    )(parts, norm2d, b)


def kernel(features, edge_index, W0, b0, W1, b1, W2, b2, num_bits, num_grad_bits):
    n, _ = features.shape
    e = edge_index.shape[1]

    epw = ((e + NW - 1) // NW + 1023) // 1024 * 1024
    e_pad = epw * NW
    rows_per_worker = epw // 128
    pad_idx = jnp.arange(e_pad - e, dtype=jnp.int32)
    src = jnp.concatenate(
        [edge_index[0], pad_idx % n]
    ).reshape(e_pad // 128, 128)
    dst = jnp.concatenate(
        [edge_index[1], n + pad_idx % (N_ACC - n)]
    ).reshape(e_pad // 128, 128)

    deg_parts = _sc_degree(dst, rows_per_worker)
    deg_nm = deg_parts[:, :n].T

    norm2d, m0 = _tc_first(deg_nm, features, W0)
    p0 = _sc_aggregate(m0, src, dst, 128, rows_per_worker)

    m1 = _tc_post_and_next(p0, norm2d, b0, W1)
    p1 = _sc_aggregate(m1, src, dst, 128, rows_per_worker)

    w2p = jnp.concatenate([W2, jnp.zeros((W2.shape[0], 128 - W2.shape[1]),
                                         jnp.float32)], axis=1)
    m2 = _tc_post_and_next(p1, norm2d, b1, w2p)
    p2 = _sc_aggregate(m2, src, dst, 128, rows_per_worker)

    return _tc_final(p2, norm2d, b2)
